# baseline (speedup 1.0000x reference)
import jax
import jax.numpy as jnp
from jax.experimental import pallas as pl
from jax.experimental.pallas import tpu as pltpu

_TOKENS = 16384
_HIDDEN = 4096
_EXPERTS = 64
_BT = 512
_NC = _TOKENS // _BT
_DEPTH = 6
_AHEAD = 4


_HB = _BT // 2


def _chunk_copies(hs_hbm, bufs, sems, c):
    slot = jax.lax.rem(c, _DEPTH)
    return [
        pltpu.make_async_copy(
            hs_hbm.at[pl.ds(c * _BT + h * _HB, _HB), :],
            bufs.at[slot, pl.ds(h * _HB, _HB), :],
            sems.at[slot, h],
        )
        for h in range(2)
    ]


def _start_chunk_copy(hs_hbm, bufs, sems, c):
    for cp in _chunk_copies(hs_hbm, bufs, sems, c):
        cp.start()


def _gate_block(thr_ref, hs_hbm, sm_ref, rw_ref, scores_ref, k_ref,
                mask_ref, sn_ref, bufs, sems):
    i = pl.program_id(0)

    @pl.when(i == 0)
    def _():
        sm = sm_ref[...]
        cnorm = jnp.sqrt(jnp.sum(sm * sm, axis=0, keepdims=True))
        sn_ref[...] = sm * (1.0 / jnp.maximum(cnorm, 1e-12))
        for c in range(_AHEAD):
            _start_chunk_copy(hs_hbm, bufs, sems, c)

    @pl.when(i + _AHEAD < _NC)
    def _():
        _start_chunk_copy(hs_hbm, bufs, sems, i + _AHEAD)

    slot = jax.lax.rem(i, _DEPTH)
    for cp in _chunk_copies(hs_hbm, bufs, sems, i):
        cp.wait()

    hs = bufs[slot]
    rnorm = jnp.sqrt(jnp.sum(hs * hs, axis=1, keepdims=True))
    hn = hs * (1.0 / jnp.maximum(rnorm, 1e-12))
    scores = jax.lax.dot_general(
        hn, sn_ref[...], (((1,), (0,)), ((), ())),
        preferred_element_type=jnp.float32)
    thr = thr_ref[0]
    mask = scores > thr
    maskf = mask.astype(jnp.float32)
    ones_ee = jnp.ones((_EXPERTS, _EXPERTS), jnp.float32)
    kf = jax.lax.dot_general(
        maskf, jnp.ones((_EXPERTS, 1), jnp.float32),
        (((1,), (0,)), ((), ())), preferred_element_type=jnp.float32)
    k_ref[...] = kf.astype(jnp.int32)
    e = jnp.where(mask, jnp.exp(scores), 0.0)
    s = jax.lax.dot_general(
        e, ones_ee, (((1,), (0,)), ((), ())),
        preferred_element_type=jnp.float32)
    rw = e / jnp.maximum(s, 1e-30)
    rw_ref[...] = jnp.where(s == 0.0, 1.0 / _EXPERTS, rw)
    scores_ref[...] = scores
    mask_ref[...] = mask


def kernel(hidden_states, sim_matrix, threshold):
    grid = (_NC,)
    out = pl.pallas_call(
        _gate_block,
        grid=grid,
        in_specs=[
            pl.BlockSpec(memory_space=pltpu.SMEM),
            pl.BlockSpec(memory_space=pl.ANY),
            pl.BlockSpec((_HIDDEN, _EXPERTS), lambda i: (0, 0)),
        ],
        out_specs=[
            pl.BlockSpec((_BT, _EXPERTS), lambda i: (i, 0)),
            pl.BlockSpec((_BT, _EXPERTS), lambda i: (i, 0)),
            pl.BlockSpec((_BT, 1), lambda i: (i, 0)),
            pl.BlockSpec((_BT, _EXPERTS), lambda i: (i, 0)),
        ],
        out_shape=[
            jax.ShapeDtypeStruct((_TOKENS, _EXPERTS), jnp.float32),
            jax.ShapeDtypeStruct((_TOKENS, _EXPERTS), jnp.float32),
            jax.ShapeDtypeStruct((_TOKENS, 1), jnp.int32),
            jax.ShapeDtypeStruct((_TOKENS, _EXPERTS), jnp.bool_),
        ],
        scratch_shapes=[
            pltpu.VMEM((_HIDDEN, _EXPERTS), jnp.float32),
            pltpu.VMEM((_DEPTH, _BT, _HIDDEN), jnp.float32),
            pltpu.SemaphoreType.DMA((_DEPTH, 2)),
        ],
        compiler_params=pltpu.CompilerParams(
            dimension_semantics=("arbitrary",),
        ),
    )(threshold, hidden_states, sim_matrix)
    routing_weights, scores, k_per_token, activated_mask = out
    return routing_weights, scores, k_per_token.reshape(_TOKENS), activated_mask


# speedup vs baseline: 1.3266x; 1.3266x over previous
import jax
import jax.numpy as jnp
from jax.experimental import pallas as pl
from jax.experimental.pallas import tpu as pltpu

_TOKENS = 16384
_HIDDEN = 4096
_EXPERTS = 64
_BT = 512
_NC = _TOKENS // _BT
_DEPTH = 6
_AHEAD = 4
_HB = _BT // 2


def _chunk_copies(hs_hbm, bufs, sems, c):
    slot = jax.lax.rem(c, _DEPTH)
    return [
        pltpu.make_async_copy(
            hs_hbm.at[pl.ds(c * _BT + h * _HB, _HB), :],
            bufs.at[slot, pl.ds(h * _HB, _HB), :],
            sems.at[slot, h],
        )
        for h in range(2)
    ]


def _start_chunk_copy(hs_hbm, bufs, sems, c):
    for cp in _chunk_copies(hs_hbm, bufs, sems, c):
        cp.start()


def _gate_block(thr_ref, hs_hbm, smt_ref, rw_ref, scores_ref, k_ref,
                mask_ref, snt_ref, bufs, sems):
    i = pl.program_id(0)

    @pl.when(i == 0)
    def _():
        smt = smt_ref[...]
        cnorm = jnp.sqrt(jnp.sum(smt * smt, axis=1, keepdims=True))
        snt_ref[...] = smt * (1.0 / jnp.maximum(cnorm, 1e-12))
        for c in range(_AHEAD):
            _start_chunk_copy(hs_hbm, bufs, sems, c)

    @pl.when(i + _AHEAD < _NC)
    def _():
        _start_chunk_copy(hs_hbm, bufs, sems, i + _AHEAD)

    slot = jax.lax.rem(i, _DEPTH)
    for cp in _chunk_copies(hs_hbm, bufs, sems, i):
        cp.wait()

    hs = bufs[slot]
    rnorm = jnp.sqrt(jnp.sum(hs * hs, axis=1, keepdims=True))
    hn = hs * (1.0 / jnp.maximum(rnorm, 1e-12))
    scores = jax.lax.dot_general(
        snt_ref[...], hn, (((1,), (1,)), ((), ())),
        preferred_element_type=jnp.float32)
    thr = thr_ref[0]
    mask = scores > thr
    maskf = mask.astype(jnp.float32)
    ones_ee = jnp.ones((_EXPERTS, _EXPERTS), jnp.float32)
    kf = jax.lax.dot_general(
        ones_ee, maskf, (((1,), (0,)), ((), ())),
        preferred_element_type=jnp.float32)
    k_ref[...] = kf[0:1, :].astype(jnp.int32)
    e = jnp.where(mask, jnp.exp(scores), 0.0)
    s = jax.lax.dot_general(
        ones_ee, e, (((1,), (0,)), ((), ())),
        preferred_element_type=jnp.float32)
    rw = e / jnp.maximum(s, 1e-30)
    rw_ref[...] = jnp.where(s == 0.0, 1.0 / _EXPERTS, rw)
    scores_ref[...] = scores
    mask_ref[...] = mask


def kernel(hidden_states, sim_matrix, threshold):
    grid = (_NC,)
    out = pl.pallas_call(
        _gate_block,
        grid=grid,
        in_specs=[
            pl.BlockSpec(memory_space=pltpu.SMEM),
            pl.BlockSpec(memory_space=pl.ANY),
            pl.BlockSpec((_EXPERTS, _HIDDEN), lambda i: (0, 0)),
        ],
        out_specs=[
            pl.BlockSpec((_EXPERTS, _BT), lambda i: (0, i)),
            pl.BlockSpec((_EXPERTS, _BT), lambda i: (0, i)),
            pl.BlockSpec((1, _BT), lambda i: (0, i)),
            pl.BlockSpec((_EXPERTS, _BT), lambda i: (0, i)),
        ],
        out_shape=[
            jax.ShapeDtypeStruct((_EXPERTS, _TOKENS), jnp.float32),
            jax.ShapeDtypeStruct((_EXPERTS, _TOKENS), jnp.float32),
            jax.ShapeDtypeStruct((1, _TOKENS), jnp.int32),
            jax.ShapeDtypeStruct((_EXPERTS, _TOKENS), jnp.bool_),
        ],
        scratch_shapes=[
            pltpu.VMEM((_EXPERTS, _HIDDEN), jnp.float32),
            pltpu.VMEM((_DEPTH, _BT, _HIDDEN), jnp.float32),
            pltpu.SemaphoreType.DMA((_DEPTH, 2)),
        ],
        compiler_params=pltpu.CompilerParams(
            dimension_semantics=("arbitrary",),
        ),
    )(threshold, hidden_states, sim_matrix.T)
    rw_t, scores_t, k_t, mask_t = out
    return (rw_t.T, scores_t.T, k_t.reshape(_TOKENS), mask_t.T)


# speedup vs baseline: 1.3405x; 1.0105x over previous
import jax
import jax.numpy as jnp
from jax.experimental import pallas as pl
from jax.experimental.pallas import tpu as pltpu

_TOKENS = 16384
_HIDDEN = 4096
_EXPERTS = 64
_BT = 512
_NC = _TOKENS // _BT
_DEPTH = 6
_AHEAD = 4
_HB = _BT // 2


def _chunk_copies(hs_hbm, bufs, sems, c):
    slot = jax.lax.rem(c, _DEPTH)
    return [
        pltpu.make_async_copy(
            hs_hbm.at[pl.ds(c * _BT + h * _HB, _HB), :],
            bufs.at[slot, pl.ds(h * _HB, _HB), :],
            sems.at[slot, h],
        )
        for h in range(2)
    ]


def _start_chunk_copy(hs_hbm, bufs, sems, c):
    for cp in _chunk_copies(hs_hbm, bufs, sems, c):
        cp.start()


def _gate_block(thr_ref, hs_hbm, smt_ref, rw_ref, scores_ref, k_ref,
                mask_ref, snt_ref, bufs, sems):
    i = pl.program_id(0)

    @pl.when(i == 0)
    def _():
        smt = smt_ref[...]
        cnorm = jnp.sqrt(jnp.sum(smt * smt, axis=1, keepdims=True))
        snt_ref[...] = smt * (1.0 / jnp.maximum(cnorm, 1e-12))
        for c in range(_AHEAD):
            _start_chunk_copy(hs_hbm, bufs, sems, c)

    @pl.when(i + _AHEAD < _NC)
    def _():
        _start_chunk_copy(hs_hbm, bufs, sems, i + _AHEAD)

    slot = jax.lax.rem(i, _DEPTH)
    for cp in _chunk_copies(hs_hbm, bufs, sems, i):
        cp.wait()

    hs = bufs[slot]
    rnorm = jnp.sqrt(jnp.sum(hs * hs, axis=1, keepdims=True))
    hn = hs * (1.0 / jnp.maximum(rnorm, 1e-12))
    scores = jax.lax.dot_general(
        snt_ref[...], hn, (((1,), (1,)), ((), ())),
        preferred_element_type=jnp.float32)
    thr = thr_ref[0]
    mask = scores > thr
    maskf = mask.astype(jnp.float32)
    ones_ee = jnp.ones((_EXPERTS, _EXPERTS), jnp.float32)
    kf = jax.lax.dot_general(
        ones_ee, maskf, (((1,), (0,)), ((), ())),
        preferred_element_type=jnp.float32)
    k_ref[...] = kf[0:1, :].astype(jnp.int32)
    e = jnp.where(mask, jnp.exp(scores), 0.0)
    s = jax.lax.dot_general(
        ones_ee, e, (((1,), (0,)), ((), ())),
        preferred_element_type=jnp.float32)
    rw = e / jnp.maximum(s, 1e-30)
    rw_ref[...] = jnp.where(s == 0.0, 1.0 / _EXPERTS, rw)
    scores_ref[...] = scores
    mask_ref[...] = mask.astype(jnp.int8)


def kernel(hidden_states, sim_matrix, threshold):
    grid = (_NC,)
    out = pl.pallas_call(
        _gate_block,
        grid=grid,
        in_specs=[
            pl.BlockSpec(memory_space=pltpu.SMEM),
            pl.BlockSpec(memory_space=pl.ANY),
            pl.BlockSpec((_EXPERTS, _HIDDEN), lambda i: (0, 0)),
        ],
        out_specs=[
            pl.BlockSpec((_EXPERTS, _BT), lambda i: (0, i)),
            pl.BlockSpec((_EXPERTS, _BT), lambda i: (0, i)),
            pl.BlockSpec((1, _BT), lambda i: (0, i)),
            pl.BlockSpec((_EXPERTS, _BT), lambda i: (0, i)),
        ],
        out_shape=[
            jax.ShapeDtypeStruct((_EXPERTS, _TOKENS), jnp.float32),
            jax.ShapeDtypeStruct((_EXPERTS, _TOKENS), jnp.float32),
            jax.ShapeDtypeStruct((1, _TOKENS), jnp.int32),
            jax.ShapeDtypeStruct((_EXPERTS, _TOKENS), jnp.int8),
        ],
        scratch_shapes=[
            pltpu.VMEM((_EXPERTS, _HIDDEN), jnp.float32),
            pltpu.VMEM((_DEPTH, _BT, _HIDDEN), jnp.float32),
            pltpu.SemaphoreType.DMA((_DEPTH, 2)),
        ],
        compiler_params=pltpu.CompilerParams(
            dimension_semantics=("arbitrary",),
        ),
    )(threshold, hidden_states, sim_matrix.T)
    rw_t, scores_t, k_t, mask_t = out
    return (rw_t.T, scores_t.T, k_t.reshape(_TOKENS),
            mask_t.T.astype(jnp.bool_))
